# baseline (speedup 1.0000x reference)
import functools

import jax
import jax.numpy as jnp
from jax import lax
from jax.experimental import pallas as pl
from jax.experimental.pallas import tpu as pltpu
from jax.experimental.pallas import tpu_sc as plsc


def _make_sc_agg(N, D, E):
    info = plsc.get_sparse_core_info()
    NC, NS, L = info.num_cores, info.num_subcores, info.num_lanes
    NW = NC * NS
    assert E % NW == 0
    e_per_w = E // NW
    K = 80
    assert e_per_w % K == 0
    n_chunks = e_per_w // K
    CB = 25
    assert n_chunks % CB == 0 and (CB - 1) % 2 == 0
    n_blocks = n_chunks // CB
    rows_per_tile = (N // NS) // 8 * 8
    tail_row0 = rows_per_tile * NS
    tail_rows = N - tail_row0
    assert tail_rows % 8 == 0 and tail_rows <= K
    DV = D // L

    mesh = plsc.VectorSubcoreMesh(core_axis_name="c", subcore_axis_name="s")

    @functools.partial(
        pl.kernel,
        mesh=mesh,
        out_type=jax.ShapeDtypeStruct((NC, N, D), jnp.float32),
        scratch_types=[
            pltpu.MemorySpace.VMEM_SHARED((N, D), jnp.float32),
            pltpu.VMEM((CB, K), jnp.int32),
            pltpu.VMEM((CB, K), jnp.int32),
            pltpu.VMEM((CB, K), jnp.float32),
            pltpu.VMEM((K, D), jnp.float32),
            pltpu.VMEM((K, D), jnp.float32),
            pltpu.VMEM((K, D), jnp.float32),
            pltpu.SemaphoreType.DMA,
            pltpu.SemaphoreType.DMA,
            pltpu.SemaphoreType.DMA,
            pltpu.SemaphoreType.DMA,
            pltpu.SemaphoreType.DMA,
            pltpu.SemaphoreType.DMA,
        ],
    )
    def sc_agg(x_hbm, src_hbm, dst_hbm, w_hbm, out_hbm,
               agg_sh, src_v, dst_v, w_v, rows0, rows1, rows2,
               gsem0, gsem1, gsem2, ssem0, ssem1, ssem2):
        c = lax.axis_index("c")
        s = lax.axis_index("s")
        wid = s * NC + c
        rows = (rows0, rows1, rows2)
        gsems = (gsem0, gsem1, gsem2)
        ssems = (ssem0, ssem1, ssem2)

        def zero_body(i, carry):
            for d in range(DV):
                rows0[i, pl.ds(d * L, L)] = jnp.zeros((L,), jnp.float32)
            return carry
        lax.fori_loop(0, K, zero_body, 0)
        row0_ = s * rows_per_tile
        n_full = rows_per_tile // K
        rem = rows_per_tile - n_full * K
        for q in range(n_full):
            pltpu.sync_copy(rows0, agg_sh.at[pl.ds(row0_ + q * K, K)])
        if rem:
            pltpu.sync_copy(rows0.at[pl.ds(0, rem)],
                            agg_sh.at[pl.ds(row0_ + n_full * K, rem)])
        if tail_rows:
            @pl.when(s == 0)
            def _zero_tail():
                pltpu.sync_copy(rows0.at[pl.ds(0, tail_rows)],
                                agg_sh.at[pl.ds(tail_row0, tail_rows)])
        plsc.subcore_barrier()

        def gather_start(ci, b):
            pltpu.async_copy(x_hbm.at[src_v.at[ci]], rows[b], gsems[b])

        def gather_wait(ci, b):
            pltpu.make_async_copy(x_hbm.at[src_v.at[ci]], rows[b],
                                  gsems[b]).wait()

        def scale(ci, b):
            def grp(g2, carry2):
                j0 = g2 * L
                w16 = w_v[ci, pl.ds(j0, L)]
                for jj in range(L):
                    wspl = jnp.full((L,), w16[jj], jnp.float32)
                    for d in range(DV):
                        rows[b][j0 + jj, pl.ds(d * L, L)] = (
                            rows[b][j0 + jj, pl.ds(d * L, L)] * wspl)
                return carry2
            lax.fori_loop(0, K // L, grp, 0)

        def scatter_start(ci, b):
            pltpu.async_copy(rows[b], agg_sh.at[dst_v.at[ci]], ssems[b],
                             add=True)

        def scatter_wait(ci, b):
            pltpu.make_async_copy(rows[b], agg_sh.at[dst_v.at[ci]],
                                  ssems[b]).wait()

        def step(ci, b, wait_scat, prefetch):
            gather_wait(ci, b)
            if wait_scat:
                scatter_wait(ci - 2, (b + 1) % 3)
            if prefetch:
                gather_start(ci + 1, (b + 1) % 3)
            scale(ci, b)
            scatter_start(ci, b)

        def block_body(bi, carry):
            pltpu.sync_copy(src_hbm.at[wid, bi], src_v)
            pltpu.sync_copy(dst_hbm.at[wid, bi], dst_v)
            pltpu.sync_copy(w_hbm.at[wid, bi], w_v)
            gather_start(jnp.int32(0), 0)
            step(jnp.int32(0), 0, False, True)
            step(jnp.int32(1), 1, False, True)

            def triple(u, carry2):
                ci = 2 + u * 3
                step(ci, 2, True, True)
                step(ci + 1, 0, True, True)
                step(ci + 2, 1, True, True)
                return carry2
            lax.fori_loop(0, (CB - 4) // 3, triple, 0)
            step(jnp.int32(CB - 2), (CB - 2) % 3, True, True)
            step(jnp.int32(CB - 1), (CB - 1) % 3, True, False)
            scatter_wait(jnp.int32(CB - 2), (CB - 2) % 3)
            scatter_wait(jnp.int32(CB - 1), (CB - 1) % 3)
            return carry
        lax.fori_loop(0, n_blocks, block_body, 0)

        plsc.subcore_barrier()
        pltpu.sync_copy(agg_sh.at[pl.ds(row0_, rows_per_tile)],
                        out_hbm.at[c, pl.ds(row0_, rows_per_tile)])
        if tail_rows:
            @pl.when(s == 0)
            def _copy_tail():
                pltpu.sync_copy(agg_sh.at[pl.ds(tail_row0, tail_rows)],
                                out_hbm.at[c, pl.ds(tail_row0, tail_rows)])

    return sc_agg, NW, n_blocks, CB, K


def _mlp_body(x_ref, p_ref, w1_ref, b1_ref, w2_ref, b2_ref, scale_ref,
              bias_ref, out_ref):
    h = x_ref[...] * scale_ref[0, 0] + p_ref[0] + p_ref[1]
    h = jnp.maximum(
        jnp.dot(h, w1_ref[...], preferred_element_type=jnp.float32)
        + b1_ref[...], 0.0)
    h = jnp.maximum(
        jnp.dot(h, w2_ref[...], preferred_element_type=jnp.float32)
        + b2_ref[...], 0.0)
    out_ref[...] = jnp.maximum(h + bias_ref[...], 0.0)


def _mlp(x, partials, W1, b1, W2, b2, scale, bias):
    N, D = x.shape
    BN = 1000
    assert N % BN == 0
    grid = (N // BN,)
    return pl.pallas_call(
        _mlp_body,
        grid=grid,
        in_specs=[
            pl.BlockSpec((BN, D), lambda i: (i, 0)),
            pl.BlockSpec((2, BN, D), lambda i: (0, i, 0)),
            pl.BlockSpec((D, D), lambda i: (0, 0)),
            pl.BlockSpec((1, D), lambda i: (0, 0)),
            pl.BlockSpec((D, D), lambda i: (0, 0)),
            pl.BlockSpec((1, D), lambda i: (0, 0)),
            pl.BlockSpec(memory_space=pltpu.SMEM),
            pl.BlockSpec((1, D), lambda i: (0, 0)),
        ],
        out_specs=pl.BlockSpec((BN, D), lambda i: (i, 0)),
        out_shape=jax.ShapeDtypeStruct((N, D), jnp.float32),
    )(x, partials, W1, b1, W2, b2, scale, bias)


def kernel(x, edge_index, edge_weight, W1, b1, W2, b2, eps, bias):
    N, D = x.shape
    E = edge_index.shape[1]
    sc_agg, NW, n_blocks, CB, K = _make_sc_agg(N, D, E)
    src = edge_index[0].reshape(NW, n_blocks, CB, K)
    dst = edge_index[1].reshape(NW, n_blocks, CB, K)
    w = edge_weight.reshape(NW, n_blocks, CB, K)
    partials = sc_agg(x, src, dst, w)
    scale = (1.0 + eps[0]).reshape(1, 1)
    return _mlp(x, partials, W1, b1.reshape(1, D), W2, b2.reshape(1, D),
                scale, bias.reshape(1, D))


# speedup vs baseline: 1.1706x; 1.1706x over previous
import functools

import jax
import jax.numpy as jnp
from jax import lax
from jax.experimental import pallas as pl
from jax.experimental.pallas import tpu as pltpu
from jax.experimental.pallas import tpu_sc as plsc


def _make_sc_agg(N, D, E):
    info = plsc.get_sparse_core_info()
    NC, NS, L = info.num_cores, info.num_subcores, info.num_lanes
    NW = NC * NS
    assert E % NW == 0
    e_per_w = E // NW
    K = 80
    assert e_per_w % K == 0
    n_chunks = e_per_w // K
    CB = 25
    assert n_chunks % CB == 0 and (CB - 4) % 3 == 0 and CB >= 5
    n_blocks = n_chunks // CB
    rows_per_tile = (N // NS) // 8 * 8
    tail_row0 = rows_per_tile * NS
    tail_rows = N - tail_row0
    assert tail_rows % 8 == 0 and tail_rows <= K
    DV = D // L

    mesh = plsc.VectorSubcoreMesh(core_axis_name="c", subcore_axis_name="s")

    @functools.partial(
        pl.kernel,
        mesh=mesh,
        out_type=jax.ShapeDtypeStruct((NC, N, D), jnp.float32),
        scratch_types=[
            pltpu.MemorySpace.VMEM_SHARED((N, D), jnp.float32),
            pltpu.VMEM((CB, K), jnp.int32),
            pltpu.VMEM((CB, K), jnp.int32),
            pltpu.VMEM((CB, K), jnp.float32),
            pltpu.VMEM((K, D), jnp.float32),
            pltpu.VMEM((K, D), jnp.float32),
            pltpu.VMEM((K, D), jnp.float32),
            pltpu.SemaphoreType.DMA,
            pltpu.SemaphoreType.DMA,
            pltpu.SemaphoreType.DMA,
        ],
    )
    def sc_agg(x_hbm, src_hbm, dst_hbm, w_hbm, out_hbm,
               agg_sh, src_v, dst_v, w_v, rows0, rows1, rows2,
               gsem0, gsem1, gsem2):
        c = lax.axis_index("c")
        s = lax.axis_index("s")
        wid = s * NC + c
        rows = (rows0, rows1, rows2)
        gsems = (gsem0, gsem1, gsem2)

        def zero_body(i, carry):
            for d in range(DV):
                rows0[i, pl.ds(d * L, L)] = jnp.zeros((L,), jnp.float32)
            return carry
        lax.fori_loop(0, K, zero_body, 0)
        row0_ = s * rows_per_tile
        n_full = rows_per_tile // K
        rem = rows_per_tile - n_full * K
        for q in range(n_full):
            pltpu.sync_copy(rows0, agg_sh.at[pl.ds(row0_ + q * K, K)])
        if rem:
            pltpu.sync_copy(rows0.at[pl.ds(0, rem)],
                            agg_sh.at[pl.ds(row0_ + n_full * K, rem)])
        if tail_rows:
            @pl.when(s == 0)
            def _zero_tail():
                pltpu.sync_copy(rows0.at[pl.ds(0, tail_rows)],
                                agg_sh.at[pl.ds(tail_row0, tail_rows)])
        plsc.subcore_barrier()

        def gather_start(ci, b):
            pltpu.async_copy(x_hbm.at[src_v.at[ci]], rows[b], gsems[b])

        def gather_wait(ci, b):
            pltpu.make_async_copy(x_hbm.at[src_v.at[ci]], rows[b],
                                  gsems[b]).wait()

        def scale(ci, b):
            def grp(g2, carry2):
                j0 = g2 * L
                w16 = w_v[ci, pl.ds(j0, L)]
                for jj in range(L):
                    wspl = jnp.full((L,), w16[jj], jnp.float32)
                    for d in range(DV):
                        rows[b][j0 + jj, pl.ds(d * L, L)] = (
                            rows[b][j0 + jj, pl.ds(d * L, L)] * wspl)
                return carry2
            lax.fori_loop(0, K // L, grp, 0)

        def scatter_sync(ci, b):
            pltpu.sync_copy(rows[b], agg_sh.at[dst_v.at[ci]], add=True)

        def step(ci, b, prefetch):
            gather_wait(ci, b)
            if prefetch:
                gather_start(ci + 2, (b + 2) % 3)
            scale(ci, b)
            scatter_sync(ci, b)

        def block_body(bi, carry):
            pltpu.sync_copy(src_hbm.at[wid, bi], src_v)
            pltpu.sync_copy(dst_hbm.at[wid, bi], dst_v)
            pltpu.sync_copy(w_hbm.at[wid, bi], w_v)
            gather_start(jnp.int32(0), 0)
            gather_start(jnp.int32(1), 1)

            def triple(u, carry2):
                ci = u * 3
                step(ci, 0, True)
                step(ci + 1, 1, True)
                step(ci + 2, 2, True)
                return carry2
            lax.fori_loop(0, (CB - 4) // 3, triple, 0)
            step(jnp.int32(CB - 4), (CB - 4) % 3, True)
            step(jnp.int32(CB - 3), (CB - 3) % 3, True)
            step(jnp.int32(CB - 2), (CB - 2) % 3, False)
            step(jnp.int32(CB - 1), (CB - 1) % 3, False)
            return carry
        lax.fori_loop(0, n_blocks, block_body, 0)

        plsc.subcore_barrier()
        pltpu.sync_copy(agg_sh.at[pl.ds(row0_, rows_per_tile)],
                        out_hbm.at[c, pl.ds(row0_, rows_per_tile)])
        if tail_rows:
            @pl.when(s == 0)
            def _copy_tail():
                pltpu.sync_copy(agg_sh.at[pl.ds(tail_row0, tail_rows)],
                                out_hbm.at[c, pl.ds(tail_row0, tail_rows)])

    return sc_agg, NW, n_blocks, CB, K


def _mlp_body(x_ref, p_ref, w1_ref, b1_ref, w2_ref, b2_ref, scale_ref,
              bias_ref, out_ref):
    h = x_ref[...] * scale_ref[0, 0] + p_ref[0] + p_ref[1]
    h = jnp.maximum(
        jnp.dot(h, w1_ref[...], preferred_element_type=jnp.float32)
        + b1_ref[...], 0.0)
    h = jnp.maximum(
        jnp.dot(h, w2_ref[...], preferred_element_type=jnp.float32)
        + b2_ref[...], 0.0)
    out_ref[...] = jnp.maximum(h + bias_ref[...], 0.0)


def _mlp(x, partials, W1, b1, W2, b2, scale, bias):
    N, D = x.shape
    BN = 1000
    assert N % BN == 0
    grid = (N // BN,)
    return pl.pallas_call(
        _mlp_body,
        grid=grid,
        in_specs=[
            pl.BlockSpec((BN, D), lambda i: (i, 0)),
            pl.BlockSpec((2, BN, D), lambda i: (0, i, 0)),
            pl.BlockSpec((D, D), lambda i: (0, 0)),
            pl.BlockSpec((1, D), lambda i: (0, 0)),
            pl.BlockSpec((D, D), lambda i: (0, 0)),
            pl.BlockSpec((1, D), lambda i: (0, 0)),
            pl.BlockSpec(memory_space=pltpu.SMEM),
            pl.BlockSpec((1, D), lambda i: (0, 0)),
        ],
        out_specs=pl.BlockSpec((BN, D), lambda i: (i, 0)),
        out_shape=jax.ShapeDtypeStruct((N, D), jnp.float32),
    )(x, partials, W1, b1, W2, b2, scale, bias)


def kernel(x, edge_index, edge_weight, W1, b1, W2, b2, eps, bias):
    N, D = x.shape
    E = edge_index.shape[1]
    sc_agg, NW, n_blocks, CB, K = _make_sc_agg(N, D, E)
    src = edge_index[0].reshape(NW, n_blocks, CB, K)
    dst = edge_index[1].reshape(NW, n_blocks, CB, K)
    w = edge_weight.reshape(NW, n_blocks, CB, K)
    partials = sc_agg(x, src, dst, w)
    scale = (1.0 + eps[0]).reshape(1, 1)
    return _mlp(x, partials, W1, b1.reshape(1, D), W2, b2.reshape(1, D),
                scale, bias.reshape(1, D))
